# baseline (speedup 1.0000x reference)
import functools

import jax
import jax.numpy as jnp
from jax import lax
from jax.experimental import pallas as pl
from jax.experimental.pallas import tpu as pltpu
from jax.experimental.pallas import tpu_sc as plsc

VOCAB = 1000000
EMBED = 64
B = 16384
H1 = 128
H2 = 64

_NC = 2
_NS = 16
_NW = _NC * _NS
_BPW = B // _NW
_IC = _BPW // 128

_TW = 8192


def _transpose_body(et_ref, out_ref):
    out_ref[...] = et_ref[...].T


def _transpose_table(embt):
    grid = (VOCAB + _TW - 1) // _TW
    return pl.pallas_call(
        _transpose_body,
        grid=(grid,),
        in_specs=[pl.BlockSpec((EMBED, _TW), lambda i: (0, i))],
        out_specs=pl.BlockSpec((_TW, EMBED), lambda i: (i, 0)),
        out_shape=jax.ShapeDtypeStruct((VOCAB, EMBED), jnp.float32),
    )(embt)


def _make_sc_gather():
    mesh = plsc.VectorSubcoreMesh(core_axis_name="c", subcore_axis_name="s")

    @functools.partial(
        pl.kernel,
        mesh=mesh,
        out_type=(
            jax.ShapeDtypeStruct((B, EMBED), jnp.float32),
            jax.ShapeDtypeStruct((B // 128, 128), jnp.float32),
        ),
        scratch_types=[
            pltpu.VMEM((_BPW // 16, 16), jnp.int32),
            pltpu.VMEM((_IC, 128), jnp.int32),
            pltpu.VMEM((_BPW, EMBED), jnp.float32),
            pltpu.VMEM((_IC, 128), jnp.float32),
            pltpu.SemaphoreType.DMA,
            pltpu.SemaphoreType.DMA,
        ],
    )
    def gather_kernel(idx16_hbm, idxr_hbm, emb_hbm, fcw_hbm, rows_out,
                      lin_out, idx16_v, idxr_v, rows_v, lin_v, sem_rows,
                      sem_lin):
        wid = lax.axis_index("s") * _NC + lax.axis_index("c")
        base = wid * _BPW
        pltpu.sync_copy(idx16_hbm.at[pl.ds(wid * (_BPW // 16), _BPW // 16)],
                        idx16_v)
        pltpu.sync_copy(idxr_hbm.at[pl.ds(wid * _IC, _IC)], idxr_v)
        fc_copies = []
        for j in range(_IC):
            fc_copies.append(pltpu.async_copy(
                fcw_hbm.at[idxr_v.at[j]], lin_v.at[j], sem_lin))
        def fire(g, carry):
            v = idx16_v[g]
            for l in range(16):
                row = v[l]
                pltpu.async_copy(emb_hbm.at[pl.ds(row, 1)],
                                 rows_v.at[pl.ds(g * 16 + l, 1)], sem_rows)
            return carry
        lax.fori_loop(0, _BPW // 16, fire, 0)
        pltpu.make_async_copy(emb_hbm.at[pl.ds(0, _BPW)], rows_v,
                              sem_rows).wait()
        pltpu.sync_copy(rows_v, rows_out.at[pl.ds(base, _BPW)])
        for cp in fc_copies:
            cp.wait()
        pltpu.sync_copy(lin_v, lin_out.at[pl.ds(wid * _IC, _IC)])

    return gather_kernel


_sc_gather = _make_sc_gather()


def _mlp_body(e_ref, linv_ref, w1t_ref, g1_ref, be1_ref,
              w2t_ref, g2_ref, be2_ref, wo_ref, c_ref, out_ref):
    e = e_ref[...]
    z1 = jnp.dot(e, w1t_ref[...], preferred_element_type=jnp.float32)
    m1 = jnp.mean(z1, axis=0, keepdims=True)
    v1 = jnp.mean(z1 * z1, axis=0, keepdims=True) - m1 * m1
    a1 = jnp.maximum(
        (z1 - m1) * lax.rsqrt(v1 + 1e-5) * g1_ref[...] + be1_ref[...], 0.0)
    z2 = jnp.dot(a1, w2t_ref[...], preferred_element_type=jnp.float32)
    m2 = jnp.mean(z2, axis=0, keepdims=True)
    v2 = jnp.mean(z2 * z2, axis=0, keepdims=True) - m2 * m2
    a2 = jnp.maximum(
        (z2 - m2) * lax.rsqrt(v2 + 1e-5) * g2_ref[...] + be2_ref[...], 0.0)
    mlp = jnp.sum(a2 * wo_ref[...], axis=1, keepdims=True)
    out_ref[...] = jax.nn.sigmoid(linv_ref[...] + mlp + c_ref[0])


def kernel(x, emb, fc_w, fc_b, w1, b1, g1, be1, w2, b2, g2, be2, wo, bo):
    xi = x.astype(jnp.int32)
    idx16 = jnp.reshape(xi, (B // 16, 16))
    idxr = jnp.reshape(xi, (B // 128, 128))
    emb_rm = _transpose_table(emb.T)
    e, lin2d = _sc_gather(idx16, idxr, emb_rm, jnp.reshape(fc_w, (VOCAB,)))
    lin = jnp.reshape(lin2d, (B, 1))
    c = (fc_b + bo).astype(jnp.float32)
    out2d = pl.pallas_call(
        _mlp_body,
        out_shape=jax.ShapeDtypeStruct((B, 1), jnp.float32),
        in_specs=[pl.BlockSpec()] * 9
        + [pl.BlockSpec(memory_space=pltpu.SMEM)],
    )(e, lin, w1.T, jnp.reshape(g1, (1, H1)), jnp.reshape(be1, (1, H1)),
      w2.T, jnp.reshape(g2, (1, H2)), jnp.reshape(be2, (1, H2)),
      jnp.reshape(wo, (1, H2)), c)
    return jnp.reshape(out2d, (B,))
